# baseline (speedup 1.0000x reference)
import functools
import jax
import jax.numpy as jnp
from jax import lax
from jax.experimental import pallas as pl
from jax.experimental.pallas import tpu as pltpu
from jax.experimental.pallas import tpu_sc as plsc

B, S, D = 2, 2048, 1024
E, K, I = 8, 2, 512
SH = 2 * 512
T = B * S

TM = 512
NT = T * K // TM + E
NROWS = NT * TM
NW = 32
TPW = T // NW
RC = 32
NCH = TPW // RC

_dims_nt = (((1,), (1,)), ((), ()))


def _gate_body(x_ref, gw_ref, gb_ref, w_ref, idx_ref, coeff_ref, d0_ref,
               d1_ref, te_ref, pos_ref, mask_ref):
    x = x_ref[...]
    scores = lax.dot_general(x, gw_ref[...], _dims_nt,
                             preferred_element_type=jnp.float32) + gb_ref[...]
    m = jnp.max(scores, axis=1, keepdims=True)
    ex = jnp.exp(scores - m)
    probs = ex / jnp.sum(ex, axis=1, keepdims=True)
    w_ref[...] = probs
    iota8 = lax.broadcasted_iota(jnp.int32, (T, E), 1)
    m0 = jnp.max(probs, axis=1, keepdims=True)
    i0 = jnp.min(jnp.where(probs == m0, iota8, E), axis=1)
    pm = jnp.where(iota8 == i0[:, None], -jnp.inf, probs)
    m1 = jnp.max(pm, axis=1, keepdims=True)
    i1 = jnp.min(jnp.where(pm == m1, iota8, E), axis=1)
    idx_ref[...] = jnp.concatenate([i0[:, None], i1[:, None]], axis=1)
    coeff_ref[...] = jnp.sum(probs, axis=1, keepdims=True)

    onehot0 = iota8 == i0[:, None]
    onehot1 = iota8 == i1[:, None]
    mask_ref[...] = (onehot0 | onehot1).astype(jnp.float32)
    C = 256
    NCk = T // C
    r_io = lax.broadcasted_iota(jnp.int32, (C, C), 0)
    c_io = lax.broadcasted_iota(jnp.int32, (C, C), 1)
    tril = (c_io < r_io).astype(jnp.bfloat16)

    intra = []
    totals = []
    for c in range(NCk):
        mc = mask_ref[c * C:(c + 1) * C, :]
        intra.append(lax.dot_general(tril, mc.astype(jnp.bfloat16),
                                     (((1,), (0,)), ((), ())),
                                     preferred_element_type=jnp.float32))
        totals.append(jnp.sum(mc, axis=0, keepdims=True))
    tot = jnp.concatenate(totals, axis=0)
    k_r = lax.broadcasted_iota(jnp.int32, (NCk, NCk), 0)
    k_c = lax.broadcasted_iota(jnp.int32, (NCk, NCk), 1)
    trilk = (k_c < k_r).astype(jnp.float32)
    coffs = lax.dot_general(trilk, tot, (((1,), (0,)), ((), ())),
                            preferred_element_type=jnp.float32)
    for c in range(NCk):
        pos_ref[c * C:(c + 1) * C, :] = intra[c] + coffs[c:c + 1, :]
    counts = coffs[NCk - 1:NCk, :] + tot[NCk - 1:NCk, :]
    pc = jnp.floor((counts + (TM - 1)) / TM) * TM
    e_r = lax.broadcasted_iota(jnp.int32, (E, E), 0)
    e_c = lax.broadcasted_iota(jnp.int32, (E, E), 1)
    tril8 = (e_c < e_r).astype(jnp.float32)
    off = lax.dot_general(pc, tril8, _dims_nt,
                          preferred_element_type=jnp.float32)
    ends = ((off + pc) * (1.0 / TM)).astype(jnp.int32)
    t_io = lax.broadcasted_iota(jnp.int32, (NT, E), 0)
    te = jnp.sum((t_io >= ends).astype(jnp.int32), axis=1)
    te_ref[...] = jnp.minimum(te, E - 1)[None, :]
    slot = off + pos_ref[...]
    d0 = jnp.sum(jnp.where(onehot0, slot, 0.0), axis=1)
    d1 = jnp.sum(jnp.where(onehot1, slot, 0.0), axis=1)
    d0_ref[...] = jnp.reshape(d0.astype(jnp.int32), (NW, TPW))
    d1_ref[...] = jnp.reshape(d1.astype(jnp.int32), (NW, TPW))


def _gate(xf, gw, gb):
    return pl.pallas_call(
        _gate_body,
        out_shape=(
            jax.ShapeDtypeStruct((T, E), jnp.float32),
            jax.ShapeDtypeStruct((T, K), jnp.int32),
            jax.ShapeDtypeStruct((T, 1), jnp.float32),
            jax.ShapeDtypeStruct((NW, TPW), jnp.int32),
            jax.ShapeDtypeStruct((NW, TPW), jnp.int32),
            jax.ShapeDtypeStruct((1, NT), jnp.int32),
        ),
        scratch_shapes=[pltpu.VMEM((T, E), jnp.float32),
                        pltpu.VMEM((T, E), jnp.float32)],
    )(xf, gw, gb.reshape(1, E))


def _sc_dispatch(xf, d30, d31):
    mesh = plsc.VectorSubcoreMesh(core_axis_name="c", subcore_axis_name="s")

    @functools.partial(
        pl.kernel, mesh=mesh,
        out_type=jax.ShapeDtypeStruct((NROWS, D), jnp.float32),
        scratch_types=[pltpu.VMEM((NCH, RC), jnp.int32),
                       pltpu.VMEM((NCH, RC), jnp.int32),
                       pltpu.VMEM((RC, D), jnp.float32),
                       pltpu.VMEM((RC, D), jnp.float32),
                       pltpu.SemaphoreType.DMA,
                       pltpu.SemaphoreType.DMA,
                       pltpu.SemaphoreType.DMA,
                       pltpu.SemaphoreType.DMA],
    )
    def run(x_hbm, d30_hbm, d31_hbm, xs_hbm, i0_v, i1_v, b0, b1, l0, l1, s0,
            s1):
        wid = lax.axis_index("s") * 2 + lax.axis_index("c")
        pltpu.sync_copy(d30_hbm.at[wid], i0_v)
        pltpu.sync_copy(d31_hbm.at[wid], i1_v)
        bufs = (b0, b1)
        lsem = (l0, l1)
        ssem = (s0, s1)
        lds = [None] * NCH
        scs = [None] * NCH

        def scatter(c):
            b = bufs[c % 2]
            lds[c].wait()
            h0 = pltpu.async_copy(b, xs_hbm.at[i0_v.at[c]], ssem[c % 2])
            h1 = pltpu.async_copy(b, xs_hbm.at[i1_v.at[c]], ssem[c % 2])
            scs[c] = (h0, h1)

        for c in range(NCH):
            if c >= 2:
                scs[c - 2][0].wait()
                scs[c - 2][1].wait()
            base = wid * TPW + c * RC
            lds[c] = pltpu.async_copy(x_hbm.at[pl.ds(base, RC)], bufs[c % 2],
                                      lsem[c % 2])
            if c >= 1:
                scatter(c - 1)
        scatter(NCH - 1)
        scs[NCH - 2][0].wait()
        scs[NCH - 2][1].wait()
        scs[NCH - 1][0].wait()
        scs[NCH - 1][1].wait()

    return run(xf, d30, d31)


def _sc_combine(o_rows, d30, d31):
    mesh = plsc.VectorSubcoreMesh(core_axis_name="c", subcore_axis_name="s")

    @functools.partial(
        pl.kernel, mesh=mesh,
        out_type=(jax.ShapeDtypeStruct((T, D), jnp.float32),
                  jax.ShapeDtypeStruct((T, D), jnp.float32)),
        scratch_types=[pltpu.VMEM((NCH, RC), jnp.int32),
                       pltpu.VMEM((NCH, RC), jnp.int32),
                       pltpu.VMEM((RC, D), jnp.float32),
                       pltpu.VMEM((RC, D), jnp.float32),
                       pltpu.SemaphoreType.DMA,
                       pltpu.SemaphoreType.DMA,
                       pltpu.SemaphoreType.DMA,
                       pltpu.SemaphoreType.DMA],
    )
    def run(o_hbm, d30_hbm, d31_hbm, y0_hbm, y1_hbm, i0_v, i1_v, b0, b1, g0,
            g1, s0, s1):
        wid = lax.axis_index("s") * 2 + lax.axis_index("c")
        pltpu.sync_copy(d30_hbm.at[wid], i0_v)
        pltpu.sync_copy(d31_hbm.at[wid], i1_v)
        bufs = (b0, b1)
        gsem = (g0, g1)
        ssem = (s0, s1)
        NP = 2 * NCH
        ghs = [None] * NP
        shs = [None] * NP

        def store(p):
            c, k = p // 2, p % 2
            yh = y0_hbm if k == 0 else y1_hbm
            ghs[p].wait()
            shs[p] = pltpu.async_copy(
                bufs[p % 2], yh.at[pl.ds(wid * TPW + c * RC, RC)],
                ssem[p % 2])

        for p in range(NP):
            c, k = p // 2, p % 2
            iv = i0_v if k == 0 else i1_v
            if p >= 2:
                shs[p - 2].wait()
            ghs[p] = pltpu.async_copy(o_hbm.at[iv.at[c]], bufs[p % 2],
                                      gsem[p % 2])
            if p >= 1:
                store(p - 1)
        store(NP - 1)
        shs[NP - 2].wait()
        shs[NP - 1].wait()

    return run(o_rows, d30, d31)


def _group_body(meta_ref, xs_ref, w1_ref, b1_ref, w3_ref, b3_ref, w2_ref,
                b2_ref, o_ref):
    x = xs_ref[...].astype(jnp.bfloat16)
    h1 = lax.dot_general(x, w1_ref[0].astype(jnp.bfloat16), _dims_nt,
                         preferred_element_type=jnp.float32) + b1_ref[0]
    h3 = lax.dot_general(x, w3_ref[0].astype(jnp.bfloat16), _dims_nt,
                         preferred_element_type=jnp.float32) + b3_ref[0]
    h = (jax.nn.silu(h1) * h3).astype(jnp.bfloat16)
    o_ref[...] = lax.dot_general(h, w2_ref[0].astype(jnp.bfloat16), _dims_nt,
                                 preferred_element_type=jnp.float32) + b2_ref[0]


def _grouped_matmul(tile_expert, xs, ew1, eb1, ew3, eb3, ew2, eb2):
    grid_spec = pltpu.PrefetchScalarGridSpec(
        num_scalar_prefetch=1,
        grid=(NT,),
        in_specs=[
            pl.BlockSpec((TM, D), lambda i, m: (i, 0)),
            pl.BlockSpec((1, I, D), lambda i, m: (m[0, i], 0, 0)),
            pl.BlockSpec((1, 1, I), lambda i, m: (m[0, i], 0, 0)),
            pl.BlockSpec((1, I, D), lambda i, m: (m[0, i], 0, 0)),
            pl.BlockSpec((1, 1, I), lambda i, m: (m[0, i], 0, 0)),
            pl.BlockSpec((1, D, I), lambda i, m: (m[0, i], 0, 0)),
            pl.BlockSpec((1, 1, D), lambda i, m: (m[0, i], 0, 0)),
        ],
        out_specs=pl.BlockSpec((TM, D), lambda i, m: (i, 0)),
    )
    return pl.pallas_call(
        _group_body,
        grid_spec=grid_spec,
        out_shape=jax.ShapeDtypeStruct((NROWS, D), jnp.float32),
    )(tile_expert, xs, ew1, eb1.reshape(E, 1, I), ew3, eb3.reshape(E, 1, I),
      ew2, eb2.reshape(E, 1, D))


_TMS = 512


def _shared_body(x_ref, sw1_ref, sb1_ref, sw3_ref, sb3_ref, sw2_ref, sb2_ref,
                 z_ref):
    x = x_ref[...].astype(jnp.bfloat16)
    h1 = lax.dot_general(x, sw1_ref[...].astype(jnp.bfloat16), _dims_nt,
                         preferred_element_type=jnp.float32) + sb1_ref[...]
    h3 = lax.dot_general(x, sw3_ref[...].astype(jnp.bfloat16), _dims_nt,
                         preferred_element_type=jnp.float32) + sb3_ref[...]
    h = (jax.nn.silu(h1) * h3).astype(jnp.bfloat16)
    z_ref[...] = lax.dot_general(h, sw2_ref[...].astype(jnp.bfloat16), _dims_nt,
                                 preferred_element_type=jnp.float32) + sb2_ref[...]


def _shared(xf, sw1, sb1, sw3, sb3, sw2, sb2, blk0, nblk):
    return pl.pallas_call(
        _shared_body,
        grid=(nblk,),
        in_specs=[
            pl.BlockSpec((_TMS, D), lambda i: (i + blk0, 0)),
            pl.BlockSpec((SH, D), lambda i: (0, 0)),
            pl.BlockSpec((1, SH), lambda i: (0, 0)),
            pl.BlockSpec((SH, D), lambda i: (0, 0)),
            pl.BlockSpec((1, SH), lambda i: (0, 0)),
            pl.BlockSpec((D, SH), lambda i: (0, 0)),
            pl.BlockSpec((1, D), lambda i: (0, 0)),
        ],
        out_specs=pl.BlockSpec((_TMS, D), lambda i: (i, 0)),
        out_shape=jax.ShapeDtypeStruct((nblk * _TMS, D), jnp.float32),
    )(xf, sw1, sb1.reshape(1, SH), sw3, sb3.reshape(1, SH), sw2,
      sb2.reshape(1, D))


def _epilogue_body(y0_ref, y1_ref, coeff_ref, z_ref, o_ref):
    o_ref[...] = (y0_ref[...] + y1_ref[...]) * coeff_ref[...] + z_ref[...]


def _epilogue(y0, y1, coeff, z):
    return pl.pallas_call(
        _epilogue_body,
        grid=(T // _TMS,),
        in_specs=[
            pl.BlockSpec((_TMS, D), lambda i: (i, 0)),
            pl.BlockSpec((_TMS, D), lambda i: (i, 0)),
            pl.BlockSpec((_TMS, 1), lambda i: (i, 0)),
            pl.BlockSpec((_TMS, D), lambda i: (i, 0)),
        ],
        out_specs=pl.BlockSpec((_TMS, D), lambda i: (i, 0)),
        out_shape=jax.ShapeDtypeStruct((T, D), jnp.float32),
    )(y0, y1, coeff, z)


@jax.jit
def kernel(x, gw, gb, ew1, eb1, ew2, eb2, ew3, eb3, sw1, sb1, sw2, sb2, sw3,
           sb3):
    shape = x.shape
    xf = x.reshape(-1, D)
    weights, indices, coeff, dest0, dest1, tile_expert = _gate(xf, gw, gb)

    d30 = dest0.reshape(NW, NCH, RC)
    d31 = dest1.reshape(NW, NCH, RC)

    xs = _sc_dispatch(xf, d30, d31)
    z = _shared(xf, sw1, sb1, sw3, sb3, sw2, sb2, 0, T // _TMS)
    o_rows = _grouped_matmul(tile_expert, xs, ew1, eb1, ew3, eb3, ew2, eb2)
    y0, y1 = _sc_combine(o_rows, d30, d31)
    out = _epilogue(y0, y1, coeff, z)
    return (weights, indices, out.reshape(shape))


# speedup vs baseline: 1.0314x; 1.0314x over previous
import functools
import jax
import jax.numpy as jnp
from jax import lax
from jax.experimental import pallas as pl
from jax.experimental.pallas import tpu as pltpu
from jax.experimental.pallas import tpu_sc as plsc

B, S, D = 2, 2048, 1024
E, K, I = 8, 2, 512
SH = 2 * 512
T = B * S

TM = 512
NT = T * K // TM + E
NROWS = NT * TM
NW = 32
TPW = T // NW
RC = 32
NCH = TPW // RC

_dims_nt = (((1,), (1,)), ((), ()))


def _gate_body(x_ref, gw_ref, gb_ref, w_ref, idx_ref, coeff_ref, d0_ref,
               d1_ref, te_ref, pos_ref, mask_ref):
    x = x_ref[...]
    scores = lax.dot_general(x, gw_ref[...], _dims_nt,
                             preferred_element_type=jnp.float32) + gb_ref[...]
    m = jnp.max(scores, axis=1, keepdims=True)
    ex = jnp.exp(scores - m)
    probs = ex / jnp.sum(ex, axis=1, keepdims=True)
    w_ref[...] = probs
    iota8 = lax.broadcasted_iota(jnp.int32, (T, E), 1)
    m0 = jnp.max(probs, axis=1, keepdims=True)
    i0 = jnp.min(jnp.where(probs == m0, iota8, E), axis=1)
    pm = jnp.where(iota8 == i0[:, None], -jnp.inf, probs)
    m1 = jnp.max(pm, axis=1, keepdims=True)
    i1 = jnp.min(jnp.where(pm == m1, iota8, E), axis=1)
    idx_ref[...] = jnp.concatenate([i0[:, None], i1[:, None]], axis=1)
    coeff_ref[...] = jnp.sum(probs, axis=1, keepdims=True)

    onehot0 = iota8 == i0[:, None]
    onehot1 = iota8 == i1[:, None]
    mask_ref[...] = (onehot0 | onehot1).astype(jnp.float32)
    C = 256
    NCk = T // C
    r_io = lax.broadcasted_iota(jnp.int32, (C, C), 0)
    c_io = lax.broadcasted_iota(jnp.int32, (C, C), 1)
    tril = (c_io < r_io).astype(jnp.bfloat16)

    intra = []
    totals = []
    for c in range(NCk):
        mc = mask_ref[c * C:(c + 1) * C, :]
        intra.append(lax.dot_general(tril, mc.astype(jnp.bfloat16),
                                     (((1,), (0,)), ((), ())),
                                     preferred_element_type=jnp.float32))
        totals.append(jnp.sum(mc, axis=0, keepdims=True))
    tot = jnp.concatenate(totals, axis=0)
    k_r = lax.broadcasted_iota(jnp.int32, (NCk, NCk), 0)
    k_c = lax.broadcasted_iota(jnp.int32, (NCk, NCk), 1)
    trilk = (k_c < k_r).astype(jnp.float32)
    coffs = lax.dot_general(trilk, tot, (((1,), (0,)), ((), ())),
                            preferred_element_type=jnp.float32)
    for c in range(NCk):
        pos_ref[c * C:(c + 1) * C, :] = intra[c] + coffs[c:c + 1, :]
    counts = coffs[NCk - 1:NCk, :] + tot[NCk - 1:NCk, :]
    pc = jnp.floor((counts + (TM - 1)) / TM) * TM
    e_r = lax.broadcasted_iota(jnp.int32, (E, E), 0)
    e_c = lax.broadcasted_iota(jnp.int32, (E, E), 1)
    tril8 = (e_c < e_r).astype(jnp.float32)
    off = lax.dot_general(pc, tril8, _dims_nt,
                          preferred_element_type=jnp.float32)
    ends = ((off + pc) * (1.0 / TM)).astype(jnp.int32)
    t_io = lax.broadcasted_iota(jnp.int32, (NT, E), 0)
    te = jnp.sum((t_io >= ends).astype(jnp.int32), axis=1)
    te_ref[...] = jnp.minimum(te, E - 1)[None, :]
    slot = off + pos_ref[...]
    d0 = jnp.sum(jnp.where(onehot0, slot, 0.0), axis=1)
    d1 = jnp.sum(jnp.where(onehot1, slot, 0.0), axis=1)
    d0_ref[...] = jnp.reshape(d0.astype(jnp.int32), (NW, TPW))
    d1_ref[...] = jnp.reshape(d1.astype(jnp.int32), (NW, TPW))


def _gate(xf, gw, gb):
    return pl.pallas_call(
        _gate_body,
        out_shape=(
            jax.ShapeDtypeStruct((T, E), jnp.float32),
            jax.ShapeDtypeStruct((T, K), jnp.int32),
            jax.ShapeDtypeStruct((T, 1), jnp.float32),
            jax.ShapeDtypeStruct((NW, TPW), jnp.int32),
            jax.ShapeDtypeStruct((NW, TPW), jnp.int32),
            jax.ShapeDtypeStruct((1, NT), jnp.int32),
        ),
        scratch_shapes=[pltpu.VMEM((T, E), jnp.float32),
                        pltpu.VMEM((T, E), jnp.float32)],
    )(xf, gw, gb.reshape(1, E))


def _sc_dispatch(xf, d30, d31):
    mesh = plsc.VectorSubcoreMesh(core_axis_name="c", subcore_axis_name="s")

    @functools.partial(
        pl.kernel, mesh=mesh,
        out_type=jax.ShapeDtypeStruct((NROWS, D), jnp.float32),
        scratch_types=[pltpu.VMEM((NCH, RC), jnp.int32),
                       pltpu.VMEM((NCH, RC), jnp.int32),
                       pltpu.VMEM((RC, D), jnp.float32),
                       pltpu.VMEM((RC, D), jnp.float32),
                       pltpu.SemaphoreType.DMA,
                       pltpu.SemaphoreType.DMA,
                       pltpu.SemaphoreType.DMA,
                       pltpu.SemaphoreType.DMA],
    )
    def run(x_hbm, d30_hbm, d31_hbm, xs_hbm, i0_v, i1_v, b0, b1, l0, l1, s0,
            s1):
        wid = lax.axis_index("s") * 2 + lax.axis_index("c")
        pltpu.sync_copy(d30_hbm.at[wid], i0_v)
        pltpu.sync_copy(d31_hbm.at[wid], i1_v)
        bufs = (b0, b1)
        lsem = (l0, l1)
        ssem = (s0, s1)
        lds = [None] * NCH
        scs = [None] * NCH

        def scatter(c):
            b = bufs[c % 2]
            lds[c].wait()
            h0 = pltpu.async_copy(b, xs_hbm.at[i0_v.at[c]], ssem[c % 2])
            h1 = pltpu.async_copy(b, xs_hbm.at[i1_v.at[c]], ssem[c % 2])
            scs[c] = (h0, h1)

        for c in range(NCH):
            if c >= 2:
                scs[c - 2][0].wait()
                scs[c - 2][1].wait()
            base = wid * TPW + c * RC
            lds[c] = pltpu.async_copy(x_hbm.at[pl.ds(base, RC)], bufs[c % 2],
                                      lsem[c % 2])
            if c >= 1:
                scatter(c - 1)
        scatter(NCH - 1)
        scs[NCH - 2][0].wait()
        scs[NCH - 2][1].wait()
        scs[NCH - 1][0].wait()
        scs[NCH - 1][1].wait()

    return run(xf, d30, d31)


def _sc_combine(o_rows, d30, d31):
    mesh = plsc.VectorSubcoreMesh(core_axis_name="c", subcore_axis_name="s")

    @functools.partial(
        pl.kernel, mesh=mesh,
        out_type=(jax.ShapeDtypeStruct((T, D), jnp.float32),
                  jax.ShapeDtypeStruct((T, D), jnp.float32)),
        scratch_types=[pltpu.VMEM((NCH, RC), jnp.int32),
                       pltpu.VMEM((NCH, RC), jnp.int32),
                       pltpu.VMEM((RC, D), jnp.float32),
                       pltpu.VMEM((RC, D), jnp.float32),
                       pltpu.SemaphoreType.DMA,
                       pltpu.SemaphoreType.DMA,
                       pltpu.SemaphoreType.DMA,
                       pltpu.SemaphoreType.DMA],
    )
    def run(o_hbm, d30_hbm, d31_hbm, y0_hbm, y1_hbm, i0_v, i1_v, b0, b1, g0,
            g1, s0, s1):
        wid = lax.axis_index("s") * 2 + lax.axis_index("c")
        pltpu.sync_copy(d30_hbm.at[wid], i0_v)
        pltpu.sync_copy(d31_hbm.at[wid], i1_v)
        bufs = (b0, b1)
        gsem = (g0, g1)
        ssem = (s0, s1)
        NP = 2 * NCH
        ghs = [None] * NP
        shs = [None] * NP

        def store(p):
            c, k = p // 2, p % 2
            yh = y0_hbm if k == 0 else y1_hbm
            ghs[p].wait()
            shs[p] = pltpu.async_copy(
                bufs[p % 2], yh.at[pl.ds(wid * TPW + c * RC, RC)],
                ssem[p % 2])

        for p in range(NP):
            c, k = p // 2, p % 2
            iv = i0_v if k == 0 else i1_v
            if p >= 2:
                shs[p - 2].wait()
            ghs[p] = pltpu.async_copy(o_hbm.at[iv.at[c]], bufs[p % 2],
                                      gsem[p % 2])
            if p >= 1:
                store(p - 1)
        store(NP - 1)
        shs[NP - 2].wait()
        shs[NP - 1].wait()

    return run(o_rows, d30, d31)


def _group_body(meta_ref, xs_ref, w1_ref, b1_ref, w3_ref, b3_ref, w2_ref,
                b2_ref, o_ref):
    x = xs_ref[...].astype(jnp.bfloat16)
    h1 = lax.dot_general(x, w1_ref[0].astype(jnp.bfloat16), _dims_nt,
                         preferred_element_type=jnp.float32) + b1_ref[0]
    h3 = lax.dot_general(x, w3_ref[0].astype(jnp.bfloat16), _dims_nt,
                         preferred_element_type=jnp.float32) + b3_ref[0]
    h = (jax.nn.silu(h1) * h3).astype(jnp.bfloat16)
    o_ref[...] = lax.dot_general(h, w2_ref[0].astype(jnp.bfloat16), _dims_nt,
                                 preferred_element_type=jnp.float32) + b2_ref[0]


def _grouped_matmul(tile_expert, xs, ew1, eb1, ew3, eb3, ew2, eb2):
    grid_spec = pltpu.PrefetchScalarGridSpec(
        num_scalar_prefetch=1,
        grid=(NT,),
        in_specs=[
            pl.BlockSpec((TM, D), lambda i, m: (i, 0)),
            pl.BlockSpec((1, I, D), lambda i, m: (m[0, i], 0, 0)),
            pl.BlockSpec((1, 1, I), lambda i, m: (m[0, i], 0, 0)),
            pl.BlockSpec((1, I, D), lambda i, m: (m[0, i], 0, 0)),
            pl.BlockSpec((1, 1, I), lambda i, m: (m[0, i], 0, 0)),
            pl.BlockSpec((1, D, I), lambda i, m: (m[0, i], 0, 0)),
            pl.BlockSpec((1, 1, D), lambda i, m: (m[0, i], 0, 0)),
        ],
        out_specs=pl.BlockSpec((TM, D), lambda i, m: (i, 0)),
    )
    return pl.pallas_call(
        _group_body,
        grid_spec=grid_spec,
        out_shape=jax.ShapeDtypeStruct((NROWS, D), jnp.float32),
    )(tile_expert, xs, ew1, eb1.reshape(E, 1, I), ew3, eb3.reshape(E, 1, I),
      ew2, eb2.reshape(E, 1, D))


_TMS = 512


def _shared_body(x_ref, sw1_ref, sb1_ref, sw3_ref, sb3_ref, sw2_ref, sb2_ref,
                 y0_ref, y1_ref, coeff_ref, o_ref):
    x = x_ref[...].astype(jnp.bfloat16)
    h1 = lax.dot_general(x, sw1_ref[...].astype(jnp.bfloat16), _dims_nt,
                         preferred_element_type=jnp.float32) + sb1_ref[...]
    h3 = lax.dot_general(x, sw3_ref[...].astype(jnp.bfloat16), _dims_nt,
                         preferred_element_type=jnp.float32) + sb3_ref[...]
    h = (jax.nn.silu(h1) * h3).astype(jnp.bfloat16)
    z = lax.dot_general(h, sw2_ref[...].astype(jnp.bfloat16), _dims_nt,
                        preferred_element_type=jnp.float32) + sb2_ref[...]
    o_ref[...] = (y0_ref[...] + y1_ref[...]) * coeff_ref[...] + z


def _shared_combine(xf, sw1, sb1, sw3, sb3, sw2, sb2, y0, y1, coeff):
    return pl.pallas_call(
        _shared_body,
        grid=(T // _TMS,),
        in_specs=[
            pl.BlockSpec((_TMS, D), lambda i: (i, 0)),
            pl.BlockSpec((SH, D), lambda i: (0, 0)),
            pl.BlockSpec((1, SH), lambda i: (0, 0)),
            pl.BlockSpec((SH, D), lambda i: (0, 0)),
            pl.BlockSpec((1, SH), lambda i: (0, 0)),
            pl.BlockSpec((D, SH), lambda i: (0, 0)),
            pl.BlockSpec((1, D), lambda i: (0, 0)),
            pl.BlockSpec((_TMS, D), lambda i: (i, 0)),
            pl.BlockSpec((_TMS, D), lambda i: (i, 0)),
            pl.BlockSpec((_TMS, 1), lambda i: (i, 0)),
        ],
        out_specs=pl.BlockSpec((_TMS, D), lambda i: (i, 0)),
        out_shape=jax.ShapeDtypeStruct((T, D), jnp.float32),
    )(xf, sw1, sb1.reshape(1, SH), sw3, sb3.reshape(1, SH), sw2,
      sb2.reshape(1, D), y0, y1, coeff)


@jax.jit
def kernel(x, gw, gb, ew1, eb1, ew2, eb2, ew3, eb3, sw1, sb1, sw2, sb2, sw3,
           sb3):
    shape = x.shape
    xf = x.reshape(-1, D)
    weights, indices, coeff, dest0, dest1, tile_expert = _gate(xf, gw, gb)

    d30 = dest0.reshape(NW, NCH, RC)
    d31 = dest1.reshape(NW, NCH, RC)

    xs = _sc_dispatch(xf, d30, d31)
    o_rows = _grouped_matmul(tile_expert, xs, ew1, eb1, ew3, eb3, ew2, eb2)
    y0, y1 = _sc_combine(o_rows, d30, d31)
    out = _shared_combine(xf, sw1, sb1, sw3, sb3, sw2, sb2, y0, y1, coeff)
    return (weights, indices, out.reshape(shape))


# speedup vs baseline: 1.0444x; 1.0126x over previous
import functools
import jax
import jax.numpy as jnp
from jax import lax
from jax.experimental import pallas as pl
from jax.experimental.pallas import tpu as pltpu
from jax.experimental.pallas import tpu_sc as plsc

B, S, D = 2, 2048, 1024
E, K, I = 8, 2, 512
SH = 2 * 512
T = B * S

TM = 512
NT = T * K // TM + E
NROWS = NT * TM
NW = 32
TPW = T // NW
RC = 32
NCH = TPW // RC

_dims_nt = (((1,), (1,)), ((), ()))


def _gate_body(x_ref, gw_ref, gb_ref, w_ref, idx_ref, coeff_ref, d0_ref,
               d1_ref, te_ref, pos_ref, mask_ref):
    x = x_ref[...]
    scores = lax.dot_general(x, gw_ref[...], _dims_nt,
                             preferred_element_type=jnp.float32) + gb_ref[...]
    m = jnp.max(scores, axis=1, keepdims=True)
    ex = jnp.exp(scores - m)
    probs = ex / jnp.sum(ex, axis=1, keepdims=True)
    w_ref[...] = probs
    iota8 = lax.broadcasted_iota(jnp.int32, (T, E), 1)
    m0 = jnp.max(probs, axis=1, keepdims=True)
    i0 = jnp.min(jnp.where(probs == m0, iota8, E), axis=1)
    pm = jnp.where(iota8 == i0[:, None], -jnp.inf, probs)
    m1 = jnp.max(pm, axis=1, keepdims=True)
    i1 = jnp.min(jnp.where(pm == m1, iota8, E), axis=1)
    idx_ref[...] = jnp.concatenate([i0[:, None], i1[:, None]], axis=1)
    coeff_ref[...] = jnp.sum(probs, axis=1, keepdims=True)

    onehot0 = iota8 == i0[:, None]
    onehot1 = iota8 == i1[:, None]
    mask_ref[...] = (onehot0 | onehot1).astype(jnp.float32)
    C = 256
    NCk = T // C
    r_io = lax.broadcasted_iota(jnp.int32, (C, C), 0)
    c_io = lax.broadcasted_iota(jnp.int32, (C, C), 1)
    tril = (c_io < r_io).astype(jnp.bfloat16)

    intra = []
    totals = []
    for c in range(NCk):
        mc = mask_ref[c * C:(c + 1) * C, :]
        intra.append(lax.dot_general(tril, mc.astype(jnp.bfloat16),
                                     (((1,), (0,)), ((), ())),
                                     preferred_element_type=jnp.float32))
        totals.append(jnp.sum(mc, axis=0, keepdims=True))
    tot = jnp.concatenate(totals, axis=0)
    k_r = lax.broadcasted_iota(jnp.int32, (NCk, NCk), 0)
    k_c = lax.broadcasted_iota(jnp.int32, (NCk, NCk), 1)
    trilk = (k_c < k_r).astype(jnp.float32)
    coffs = lax.dot_general(trilk, tot, (((1,), (0,)), ((), ())),
                            preferred_element_type=jnp.float32)
    for c in range(NCk):
        pos_ref[c * C:(c + 1) * C, :] = intra[c] + coffs[c:c + 1, :]
    counts = coffs[NCk - 1:NCk, :] + tot[NCk - 1:NCk, :]
    pc = jnp.floor((counts + (TM - 1)) / TM) * TM
    e_r = lax.broadcasted_iota(jnp.int32, (E, E), 0)
    e_c = lax.broadcasted_iota(jnp.int32, (E, E), 1)
    tril8 = (e_c < e_r).astype(jnp.float32)
    off = lax.dot_general(pc, tril8, _dims_nt,
                          preferred_element_type=jnp.float32)
    ends = ((off + pc) * (1.0 / TM)).astype(jnp.int32)
    t_io = lax.broadcasted_iota(jnp.int32, (NT, E), 0)
    te = jnp.sum((t_io >= ends).astype(jnp.int32), axis=1)
    te_ref[...] = jnp.minimum(te, E - 1)[None, :]
    slot = off + pos_ref[...]
    d0 = jnp.sum(jnp.where(onehot0, slot, 0.0), axis=1)
    d1 = jnp.sum(jnp.where(onehot1, slot, 0.0), axis=1)
    d0_ref[...] = jnp.reshape(d0.astype(jnp.int32), (NW, TPW))
    d1_ref[...] = jnp.reshape(d1.astype(jnp.int32), (NW, TPW))


def _gate(xf, gw, gb):
    return pl.pallas_call(
        _gate_body,
        out_shape=(
            jax.ShapeDtypeStruct((T, E), jnp.float32),
            jax.ShapeDtypeStruct((T, K), jnp.int32),
            jax.ShapeDtypeStruct((T, 1), jnp.float32),
            jax.ShapeDtypeStruct((NW, TPW), jnp.int32),
            jax.ShapeDtypeStruct((NW, TPW), jnp.int32),
            jax.ShapeDtypeStruct((1, NT), jnp.int32),
        ),
        scratch_shapes=[pltpu.VMEM((T, E), jnp.float32),
                        pltpu.VMEM((T, E), jnp.float32)],
    )(xf, gw, gb.reshape(1, E))


def _sc_dispatch(xf, d30, d31):
    mesh = plsc.VectorSubcoreMesh(core_axis_name="c", subcore_axis_name="s")

    @functools.partial(
        pl.kernel, mesh=mesh,
        out_type=jax.ShapeDtypeStruct((NROWS, D), jnp.float32),
        scratch_types=[pltpu.VMEM((NCH, RC), jnp.int32),
                       pltpu.VMEM((NCH, RC), jnp.int32),
                       pltpu.VMEM((RC, D), jnp.float32),
                       pltpu.VMEM((RC, D), jnp.float32),
                       pltpu.SemaphoreType.DMA,
                       pltpu.SemaphoreType.DMA,
                       pltpu.SemaphoreType.DMA,
                       pltpu.SemaphoreType.DMA],
    )
    def run(x_hbm, d30_hbm, d31_hbm, xs_hbm, i0_v, i1_v, b0, b1, l0, l1, s0,
            s1):
        wid = lax.axis_index("s") * 2 + lax.axis_index("c")
        pltpu.sync_copy(d30_hbm.at[wid], i0_v)
        pltpu.sync_copy(d31_hbm.at[wid], i1_v)
        bufs = (b0, b1)
        lsem = (l0, l1)
        ssem = (s0, s1)
        lds = [None] * NCH
        scs = [None] * NCH

        def scatter(c):
            b = bufs[c % 2]
            lds[c].wait()
            h0 = pltpu.async_copy(b, xs_hbm.at[i0_v.at[c]], ssem[c % 2])
            h1 = pltpu.async_copy(b, xs_hbm.at[i1_v.at[c]], ssem[c % 2])
            scs[c] = (h0, h1)

        for c in range(NCH):
            if c >= 2:
                scs[c - 2][0].wait()
                scs[c - 2][1].wait()
            base = wid * TPW + c * RC
            lds[c] = pltpu.async_copy(x_hbm.at[pl.ds(base, RC)], bufs[c % 2],
                                      lsem[c % 2])
            if c >= 1:
                scatter(c - 1)
        scatter(NCH - 1)
        scs[NCH - 2][0].wait()
        scs[NCH - 2][1].wait()
        scs[NCH - 1][0].wait()
        scs[NCH - 1][1].wait()

    return run(xf, d30, d31)


def _sc_combine(o_rows, d30, d31):
    mesh = plsc.VectorSubcoreMesh(core_axis_name="c", subcore_axis_name="s")

    @functools.partial(
        pl.kernel, mesh=mesh,
        out_type=(jax.ShapeDtypeStruct((T, D), jnp.float32),
                  jax.ShapeDtypeStruct((T, D), jnp.float32)),
        scratch_types=[pltpu.VMEM((NCH, RC), jnp.int32),
                       pltpu.VMEM((NCH, RC), jnp.int32),
                       pltpu.VMEM((RC, D), jnp.float32),
                       pltpu.VMEM((RC, D), jnp.float32),
                       pltpu.SemaphoreType.DMA,
                       pltpu.SemaphoreType.DMA,
                       pltpu.SemaphoreType.DMA,
                       pltpu.SemaphoreType.DMA],
    )
    def run(o_hbm, d30_hbm, d31_hbm, y0_hbm, y1_hbm, i0_v, i1_v, b0, b1, g0,
            g1, s0, s1):
        wid = lax.axis_index("s") * 2 + lax.axis_index("c")
        pltpu.sync_copy(d30_hbm.at[wid], i0_v)
        pltpu.sync_copy(d31_hbm.at[wid], i1_v)
        bufs = (b0, b1)
        gsem = (g0, g1)
        ssem = (s0, s1)
        NP = 2 * NCH
        ghs = [None] * NP
        shs = [None] * NP

        def store(p):
            c, k = p // 2, p % 2
            yh = y0_hbm if k == 0 else y1_hbm
            ghs[p].wait()
            shs[p] = pltpu.async_copy(
                bufs[p % 2], yh.at[pl.ds(wid * TPW + c * RC, RC)],
                ssem[p % 2])

        for p in range(NP):
            c, k = p // 2, p % 2
            iv = i0_v if k == 0 else i1_v
            if p >= 2:
                shs[p - 2].wait()
            ghs[p] = pltpu.async_copy(o_hbm.at[iv.at[c]], bufs[p % 2],
                                      gsem[p % 2])
            if p >= 1:
                store(p - 1)
        store(NP - 1)
        shs[NP - 2].wait()
        shs[NP - 1].wait()

    return run(o_rows, d30, d31)


def _group_body(meta_ref, xs_ref, w1_ref, b1_ref, w3_ref, b3_ref, w2_ref,
                b2_ref, o_ref):
    x = xs_ref[...].astype(jnp.bfloat16)
    h1 = lax.dot_general(x, w1_ref[0].astype(jnp.bfloat16), _dims_nt,
                         preferred_element_type=jnp.float32) + b1_ref[0]
    h3 = lax.dot_general(x, w3_ref[0].astype(jnp.bfloat16), _dims_nt,
                         preferred_element_type=jnp.float32) + b3_ref[0]
    h = (jax.nn.silu(h1) * h3).astype(jnp.bfloat16)
    o_ref[...] = lax.dot_general(h, w2_ref[0].astype(jnp.bfloat16), _dims_nt,
                                 preferred_element_type=jnp.float32) + b2_ref[0]


def _grouped_matmul(tile_expert, xs, ew1, eb1, ew3, eb3, ew2, eb2):
    grid_spec = pltpu.PrefetchScalarGridSpec(
        num_scalar_prefetch=1,
        grid=(NT,),
        in_specs=[
            pl.BlockSpec((TM, D), lambda i, m: (i, 0)),
            pl.BlockSpec((1, I, D), lambda i, m: (m[0, i], 0, 0)),
            pl.BlockSpec((1, 1, I), lambda i, m: (m[0, i], 0, 0)),
            pl.BlockSpec((1, I, D), lambda i, m: (m[0, i], 0, 0)),
            pl.BlockSpec((1, 1, I), lambda i, m: (m[0, i], 0, 0)),
            pl.BlockSpec((1, D, I), lambda i, m: (m[0, i], 0, 0)),
            pl.BlockSpec((1, 1, D), lambda i, m: (m[0, i], 0, 0)),
        ],
        out_specs=pl.BlockSpec((TM, D), lambda i, m: (i, 0)),
    )
    return pl.pallas_call(
        _group_body,
        grid_spec=grid_spec,
        out_shape=jax.ShapeDtypeStruct((NROWS, D), jnp.float32),
    )(tile_expert, xs, ew1, eb1.reshape(E, 1, I), ew3, eb3.reshape(E, 1, I),
      ew2, eb2.reshape(E, 1, D))


_TMS = 512


def _shared_h_body(x_ref, sw1_ref, sb1_ref, sw3_ref, sb3_ref, h_ref):
    x = x_ref[...].astype(jnp.bfloat16)
    h1 = lax.dot_general(x, sw1_ref[...].astype(jnp.bfloat16), _dims_nt,
                         preferred_element_type=jnp.float32) + sb1_ref[...]
    h3 = lax.dot_general(x, sw3_ref[...].astype(jnp.bfloat16), _dims_nt,
                         preferred_element_type=jnp.float32) + sb3_ref[...]
    h_ref[...] = (jax.nn.silu(h1) * h3).astype(jnp.bfloat16)


def _shared_h(xf, sw1, sb1, sw3, sb3):
    return pl.pallas_call(
        _shared_h_body,
        grid=(T // _TMS,),
        in_specs=[
            pl.BlockSpec((_TMS, D), lambda i: (i, 0)),
            pl.BlockSpec((SH, D), lambda i: (0, 0)),
            pl.BlockSpec((1, SH), lambda i: (0, 0)),
            pl.BlockSpec((SH, D), lambda i: (0, 0)),
            pl.BlockSpec((1, SH), lambda i: (0, 0)),
        ],
        out_specs=pl.BlockSpec((_TMS, SH), lambda i: (i, 0)),
        out_shape=jax.ShapeDtypeStruct((T, SH), jnp.bfloat16),
    )(xf, sw1, sb1.reshape(1, SH), sw3, sb3.reshape(1, SH))


def _shared_body(h_ref, sw2_ref, sb2_ref, y0_ref, y1_ref, coeff_ref, o_ref):
    z = lax.dot_general(h_ref[...], sw2_ref[...].astype(jnp.bfloat16),
                        _dims_nt,
                        preferred_element_type=jnp.float32) + sb2_ref[...]
    o_ref[...] = (y0_ref[...] + y1_ref[...]) * coeff_ref[...] + z


def _shared_combine(h, sw2, sb2, y0, y1, coeff):
    return pl.pallas_call(
        _shared_body,
        grid=(T // _TMS,),
        in_specs=[
            pl.BlockSpec((_TMS, SH), lambda i: (i, 0)),
            pl.BlockSpec((D, SH), lambda i: (0, 0)),
            pl.BlockSpec((1, D), lambda i: (0, 0)),
            pl.BlockSpec((_TMS, D), lambda i: (i, 0)),
            pl.BlockSpec((_TMS, D), lambda i: (i, 0)),
            pl.BlockSpec((_TMS, 1), lambda i: (i, 0)),
        ],
        out_specs=pl.BlockSpec((_TMS, D), lambda i: (i, 0)),
        out_shape=jax.ShapeDtypeStruct((T, D), jnp.float32),
    )(h, sw2, sb2.reshape(1, D), y0, y1, coeff)


@jax.jit
def kernel(x, gw, gb, ew1, eb1, ew2, eb2, ew3, eb3, sw1, sb1, sw2, sb2, sw3,
           sb3):
    shape = x.shape
    xf = x.reshape(-1, D)
    weights, indices, coeff, dest0, dest1, tile_expert = _gate(xf, gw, gb)

    d30 = dest0.reshape(NW, NCH, RC)
    d31 = dest1.reshape(NW, NCH, RC)

    xs = _sc_dispatch(xf, d30, d31)
    h = _shared_h(xf, sw1, sb1, sw3, sb3)
    o_rows = _grouped_matmul(tile_expert, xs, ew1, eb1, ew3, eb3, ew2, eb2)
    y0, y1 = _sc_combine(o_rows, d30, d31)
    out = _shared_combine(h, sw2, sb2, y0, y1, coeff)
    return (weights, indices, out.reshape(shape))
